# Initial kernel scaffold; baseline (speedup 1.0000x reference)
import jax
import jax.numpy as jnp
from jax.experimental import pallas as pl


def kernel(trend_representation, representation, keys, values):
    raise NotImplementedError("write your pallas kernel here")


# speedup vs baseline: 3.7267x; 3.7267x over previous
import functools

import jax
import jax.numpy as jnp
from jax import lax
from jax.experimental import pallas as pl
from jax.experimental.pallas import tpu as pltpu
from jax.experimental.pallas import tpu_sc as plsc

_NC = 2
_NS = 16
_NW = _NC * _NS
_LANES = 16

_TC_TILE = 512


def _argmax_body(tr_ref, keys_ref, idx_ref):
    tr = tr_ref[...]
    keys = keys_ref[...]
    s = lax.dot_general(tr, keys, (((1,), (1,)), ((), ())),
                        preferred_element_type=jnp.float32)
    m = jnp.max(s, axis=1, keepdims=True)
    ii = lax.broadcasted_iota(jnp.int32, s.shape, 1)
    cand = jnp.where(s == m, ii, s.shape[1])
    idx_ref[0, 0, :] = jnp.min(cand, axis=1)


def _tc_argmax(tr2, keys):
    t, c = tr2.shape
    m = keys.shape[0]
    grid = t // _TC_TILE
    idx3 = pl.pallas_call(
        _argmax_body,
        grid=(grid,),
        in_specs=[
            pl.BlockSpec((_TC_TILE, c), lambda i: (i, 0)),
            pl.BlockSpec((m, c), lambda i: (0, 0)),
        ],
        out_specs=pl.BlockSpec((1, 1, _TC_TILE), lambda i: (i, 0, 0)),
        out_shape=jax.ShapeDtypeStruct((grid, 1, _TC_TILE), jnp.int32),
        compiler_params=pltpu.CompilerParams(
            dimension_semantics=("arbitrary",)),
    )(tr2, keys)
    return idx3.reshape(t)


def _make_sc_mse(t, c, m, chunk):
    per_w = t // _NW
    n_chunks = per_w // chunk
    n_groups = chunk // _LANES
    mesh = plsc.VectorSubcoreMesh(core_axis_name="c", subcore_axis_name="s",
                                  num_cores=_NC, num_subcores=_NS)

    @functools.partial(
        pl.kernel,
        out_type=[jax.ShapeDtypeStruct((t,), jnp.float32),
                  jax.ShapeDtypeStruct((t,), jnp.float32)],
        mesh=mesh,
        scratch_types=[
            pltpu.VMEM((m * c,), jnp.float32),
            pltpu.VMEM((m * c,), jnp.float32),
            pltpu.VMEM((chunk * c,), jnp.float32),
            pltpu.VMEM((chunk * c,), jnp.float32),
            pltpu.VMEM((chunk,), jnp.int32),
            pltpu.VMEM((chunk,), jnp.float32),
            pltpu.VMEM((chunk,), jnp.float32),
        ],
        compiler_params=pltpu.CompilerParams(needs_layout_passes=False),
    )
    def sc_mse(tr_hbm, rep_hbm, keys_hbm, values_hbm, idx_hbm,
               outk_hbm, outv_hbm,
               keys_v, values_v, tr_v, rep_v, idx_v, outk_v, outv_v):
        wid = lax.axis_index("s") * _NC + lax.axis_index("c")
        pltpu.sync_copy(keys_hbm, keys_v)
        pltpu.sync_copy(values_hbm, values_v)
        lanes = lax.iota(jnp.int32, _LANES)

        def chunk_body(ci, carry):
            base = wid * per_w + ci * chunk
            pltpu.sync_copy(tr_hbm.at[pl.ds(base * c, chunk * c)], tr_v)
            pltpu.sync_copy(rep_hbm.at[pl.ds(base * c, chunk * c)], rep_v)
            pltpu.sync_copy(idx_hbm.at[pl.ds(base, chunk)], idx_v)

            def group_body(g, carry2):
                rows = g * _LANES + lanes
                idxv = plsc.load_gather(idx_v, [rows])
                tok_base = rows * c
                sel_base = idxv * c
                acck = jnp.zeros((_LANES,), jnp.float32)
                accv = jnp.zeros((_LANES,), jnp.float32)
                for cc in range(c):
                    trc = plsc.load_gather(tr_v, [tok_base + cc])
                    kc = plsc.load_gather(keys_v, [sel_base + cc])
                    dk = trc - kc
                    acck = acck + dk * dk
                    rc = plsc.load_gather(rep_v, [tok_base + cc])
                    vc = plsc.load_gather(values_v, [sel_base + cc])
                    dv = rc - vc
                    accv = accv + dv * dv
                plsc.store_scatter(outk_v, [rows], acck)
                plsc.store_scatter(outv_v, [rows], accv)
                return carry2

            lax.fori_loop(0, n_groups, group_body, 0)
            pltpu.sync_copy(outk_v, outk_hbm.at[pl.ds(base, chunk)])
            pltpu.sync_copy(outv_v, outv_hbm.at[pl.ds(base, chunk)])
            return carry

        lax.fori_loop(0, n_chunks, chunk_body, 0)

    return sc_mse


def kernel(trend_representation, representation, keys, values):
    b, l, c = trend_representation.shape
    m = keys.shape[0]
    t = b * l
    tr2 = trend_representation.reshape(t, c)
    rep2 = representation.reshape(t, c)
    idx = _tc_argmax(tr2, keys)
    sc_mse = _make_sc_mse(t, c, m, chunk=512)
    kg, vg = sc_mse(tr2.reshape(-1), rep2.reshape(-1),
                    keys.reshape(-1), values.reshape(-1), idx)
    return kg.reshape(b, l), vg.reshape(b, l)


# speedup vs baseline: 5.7257x; 1.5364x over previous
import functools

import jax
import jax.numpy as jnp
from jax import lax
from jax.experimental import pallas as pl
from jax.experimental.pallas import tpu as pltpu
from jax.experimental.pallas import tpu_sc as plsc

_NC = 2
_NS = 16
_NW = _NC * _NS
_LANES = 16

_TC_TILE = 512


def _argmax_body(tr_ref, keys_ref, idx_ref):
    tr = tr_ref[...]
    keys = keys_ref[...]
    s = lax.dot_general(tr, keys, (((1,), (1,)), ((), ())),
                        preferred_element_type=jnp.float32)
    m = jnp.max(s, axis=1, keepdims=True)
    ii = lax.broadcasted_iota(jnp.int32, s.shape, 1)
    cand = jnp.where(s == m, ii, s.shape[1])
    idx_ref[0, 0, :] = jnp.min(cand, axis=1)


def _tc_argmax(tr2, keys):
    t, c = tr2.shape
    m = keys.shape[0]
    grid = t // _TC_TILE
    idx3 = pl.pallas_call(
        _argmax_body,
        grid=(grid,),
        in_specs=[
            pl.BlockSpec((_TC_TILE, c), lambda i: (i, 0)),
            pl.BlockSpec((m, c), lambda i: (0, 0)),
        ],
        out_specs=pl.BlockSpec((1, 1, _TC_TILE), lambda i: (i, 0, 0)),
        out_shape=jax.ShapeDtypeStruct((grid, 1, _TC_TILE), jnp.int32),
        compiler_params=pltpu.CompilerParams(
            dimension_semantics=("arbitrary",)),
    )(tr2, keys)
    return idx3.reshape(t)


def _make_sc_mse(t, c, m, chunk):
    per_w = t // _NW
    n_chunks = per_w // chunk
    n_groups = chunk // _LANES
    mesh = plsc.VectorSubcoreMesh(core_axis_name="c", subcore_axis_name="s",
                                  num_cores=_NC, num_subcores=_NS)

    @functools.partial(
        pl.kernel,
        out_type=[jax.ShapeDtypeStruct((t,), jnp.float32),
                  jax.ShapeDtypeStruct((t,), jnp.float32)],
        mesh=mesh,
        scratch_types=[
            pltpu.VMEM((c, m), jnp.float32),
            pltpu.VMEM((c, m), jnp.float32),
            pltpu.VMEM((c, chunk), jnp.float32),
            pltpu.VMEM((c, chunk), jnp.float32),
            pltpu.VMEM((chunk,), jnp.int32),
            pltpu.VMEM((chunk,), jnp.float32),
            pltpu.VMEM((chunk,), jnp.float32),
        ],
        compiler_params=pltpu.CompilerParams(needs_layout_passes=False),
    )
    def sc_mse(tr_hbm, rep_hbm, keys_hbm, values_hbm, idx_hbm,
               outk_hbm, outv_hbm,
               keys_v, values_v, tr_v, rep_v, idx_v, outk_v, outv_v):
        wid = lax.axis_index("s") * _NC + lax.axis_index("c")
        pltpu.sync_copy(keys_hbm, keys_v)
        pltpu.sync_copy(values_hbm, values_v)
        lanes = lax.iota(jnp.int32, _LANES)

        def chunk_body(ci, carry):
            base = wid * per_w + ci * chunk
            pltpu.sync_copy(tr_hbm.at[:, pl.ds(base, chunk)], tr_v)
            pltpu.sync_copy(rep_hbm.at[:, pl.ds(base, chunk)], rep_v)
            pltpu.sync_copy(idx_hbm.at[pl.ds(base, chunk)], idx_v)

            def group_body(g, carry2):
                rows = g * _LANES + lanes
                idxv = plsc.load_gather(idx_v, [rows])
                acck = jnp.zeros((_LANES,), jnp.float32)
                accv = jnp.zeros((_LANES,), jnp.float32)
                for cc in range(c):
                    col = jnp.full((_LANES,), cc, jnp.int32)
                    trc = plsc.load_gather(tr_v, [col, rows])
                    kc = plsc.load_gather(keys_v, [col, idxv])
                    dk = trc - kc
                    acck = acck + dk * dk
                    rc = plsc.load_gather(rep_v, [col, rows])
                    vc = plsc.load_gather(values_v, [col, idxv])
                    dv = rc - vc
                    accv = accv + dv * dv
                plsc.store_scatter(outk_v, [rows], acck)
                plsc.store_scatter(outv_v, [rows], accv)
                return carry2

            lax.fori_loop(0, n_groups, group_body, 0)
            pltpu.sync_copy(outk_v, outk_hbm.at[pl.ds(base, chunk)])
            pltpu.sync_copy(outv_v, outv_hbm.at[pl.ds(base, chunk)])
            return carry

        lax.fori_loop(0, n_chunks, chunk_body, 0)

    return sc_mse


def kernel(trend_representation, representation, keys, values):
    b, l, c = trend_representation.shape
    m = keys.shape[0]
    t = b * l
    tr2 = trend_representation.reshape(t, c)
    rep2 = representation.reshape(t, c)
    idx = _tc_argmax(tr2, keys)
    sc_mse = _make_sc_mse(t, c, m, chunk=512)
    kg, vg = sc_mse(tr2.T, rep2.T, keys.T, values.T, idx)
    return kg.reshape(b, l), vg.reshape(b, l)
